# baseline (device time: 312143 ns/iter reference)
import jax
import jax.numpy as jnp
from jax import lax
from jax.experimental import pallas as pl
from jax.experimental.pallas import tpu as pltpu

N_DEV = 4
M = 2048
N = 2048
CHUNK = M // N_DEV


def kernel(x, w_mat):
    def body(x_ref, w_ref, out_ref, comm_ref, send_sems, recv_sems, credit_sem):
        my = lax.axis_index("i")
        left = lax.rem(my + N_DEV - 1, N_DEV)
        right = lax.rem(my + 1, N_DEV)

        barrier_sem = pltpu.get_barrier_semaphore()
        for nbr in (left, right):
            pl.semaphore_signal(
                barrier_sem, inc=1,
                device_id=(nbr,), device_id_type=pl.DeviceIdType.MESH,
            )
        pl.semaphore_wait(barrier_sem, 2)

        out_ref[:, :] = jnp.dot(
            x_ref[:, :], w_ref[:, :], preferred_element_type=jnp.float32
        )

        def hop(h, src_chunk, dst_chunk, accumulate):
            slot = h % 2
            if h >= 2:
                pl.semaphore_wait(credit_sem, 1)
            rdma = pltpu.make_async_remote_copy(
                src_ref=out_ref.at[pl.ds(src_chunk * CHUNK, CHUNK), :],
                dst_ref=comm_ref.at[slot],
                send_sem=send_sems.at[slot],
                recv_sem=recv_sems.at[slot],
                device_id=(right,),
                device_id_type=pl.DeviceIdType.MESH,
            )
            rdma.start()
            rdma.wait()
            if accumulate:
                out_ref[pl.ds(dst_chunk * CHUNK, CHUNK), :] += comm_ref[slot]
            else:
                out_ref[pl.ds(dst_chunk * CHUNK, CHUNK), :] = comm_ref[slot]
            if h < 4:
                pl.semaphore_signal(
                    credit_sem, inc=1,
                    device_id=(left,), device_id_type=pl.DeviceIdType.MESH,
                )

        for h in range(N_DEV - 1):
            src_chunk = lax.rem(my - h + 2 * N_DEV, N_DEV)
            dst_chunk = lax.rem(my - 1 - h + 2 * N_DEV, N_DEV)
            hop(h, src_chunk, dst_chunk, accumulate=True)

        for a in range(N_DEV - 1):
            src_chunk = lax.rem(my + 1 - a + 2 * N_DEV, N_DEV)
            dst_chunk = lax.rem(my - a + 2 * N_DEV, N_DEV)
            hop(a + N_DEV - 1, src_chunk, dst_chunk, accumulate=False)

    return pl.pallas_call(
        body,
        out_shape=jax.ShapeDtypeStruct((M, N), jnp.float32),
        in_specs=[
            pl.BlockSpec(memory_space=pltpu.VMEM),
            pl.BlockSpec(memory_space=pltpu.VMEM),
        ],
        out_specs=pl.BlockSpec(memory_space=pltpu.VMEM),
        scratch_shapes=[
            pltpu.VMEM((2, CHUNK, N), jnp.float32),
            pltpu.SemaphoreType.DMA((2,)),
            pltpu.SemaphoreType.DMA((2,)),
            pltpu.SemaphoreType.REGULAR,
        ],
        compiler_params=pltpu.CompilerParams(collective_id=0),
    )(x, w_mat)


# device time: 111612 ns/iter; 2.7967x vs baseline; 2.7967x over previous
import jax
import jax.numpy as jnp
from jax import lax
from jax.experimental import pallas as pl
from jax.experimental.pallas import tpu as pltpu

N_DEV = 4
M = 2048
N = 2048
CHUNK = M // N_DEV
HALF = N // 2


def kernel(x, w_mat):
    def body(
        x_ref, w_ref, out_ref,
        comm_r, comm_l, sb_r, sb_l,
        send_sems_r, recv_sems_r, send_sems_l, recv_sems_l,
        credit_r, credit_l,
    ):
        my = lax.axis_index("i")
        left = lax.rem(my + N_DEV - 1, N_DEV)
        right = lax.rem(my + 1, N_DEV)

        barrier_sem = pltpu.get_barrier_semaphore()
        for nbr in (left, right):
            pl.semaphore_signal(
                barrier_sem, inc=1,
                device_id=(nbr,), device_id_type=pl.DeviceIdType.MESH,
            )
        pl.semaphore_wait(barrier_sem, 2)

        out_ref[:, :] = jnp.dot(
            x_ref[:, :], w_ref[:, :], preferred_element_type=jnp.float32
        )

        def hop(h, src_r, dst_r, src_l, dst_l, accumulate):
            slot = h % 2
            sb_r[slot] = out_ref[pl.ds(src_r * CHUNK, CHUNK), :HALF].astype(
                jnp.bfloat16
            )
            sb_l[slot] = out_ref[pl.ds(src_l * CHUNK, CHUNK), HALF:].astype(
                jnp.bfloat16
            )
            if h >= 2:
                pl.semaphore_wait(credit_r, 1)
                pl.semaphore_wait(credit_l, 1)
            rdma_r = pltpu.make_async_remote_copy(
                src_ref=sb_r.at[slot],
                dst_ref=comm_r.at[slot],
                send_sem=send_sems_r.at[slot],
                recv_sem=recv_sems_r.at[slot],
                device_id=(right,),
                device_id_type=pl.DeviceIdType.MESH,
            )
            rdma_l = pltpu.make_async_remote_copy(
                src_ref=sb_l.at[slot],
                dst_ref=comm_l.at[slot],
                send_sem=send_sems_l.at[slot],
                recv_sem=recv_sems_l.at[slot],
                device_id=(left,),
                device_id_type=pl.DeviceIdType.MESH,
            )
            rdma_r.start()
            rdma_l.start()
            rdma_r.wait()
            rdma_l.wait()
            got_r = comm_r[slot].astype(jnp.float32)
            got_l = comm_l[slot].astype(jnp.float32)
            if accumulate:
                out_ref[pl.ds(dst_r * CHUNK, CHUNK), :HALF] += got_r
                out_ref[pl.ds(dst_l * CHUNK, CHUNK), HALF:] += got_l
            else:
                out_ref[pl.ds(dst_r * CHUNK, CHUNK), :HALF] = got_r
                out_ref[pl.ds(dst_l * CHUNK, CHUNK), HALF:] = got_l
            if h < 4:
                pl.semaphore_signal(
                    credit_r, inc=1,
                    device_id=(left,), device_id_type=pl.DeviceIdType.MESH,
                )
                pl.semaphore_signal(
                    credit_l, inc=1,
                    device_id=(right,), device_id_type=pl.DeviceIdType.MESH,
                )

        def mod4(v):
            return lax.rem(v + 2 * N_DEV, N_DEV)

        for h in range(N_DEV - 1):
            hop(
                h,
                mod4(my - h), mod4(my - 1 - h),
                mod4(my + h), mod4(my + 1 + h),
                accumulate=True,
            )

        for a in range(N_DEV - 1):
            hop(
                a + N_DEV - 1,
                mod4(my + 1 - a), mod4(my - a),
                mod4(my - 1 + a), mod4(my + a),
                accumulate=False,
            )

    return pl.pallas_call(
        body,
        out_shape=jax.ShapeDtypeStruct((M, N), jnp.float32),
        in_specs=[
            pl.BlockSpec(memory_space=pltpu.VMEM),
            pl.BlockSpec(memory_space=pltpu.VMEM),
        ],
        out_specs=pl.BlockSpec(memory_space=pltpu.VMEM),
        scratch_shapes=[
            pltpu.VMEM((2, CHUNK, HALF), jnp.bfloat16),
            pltpu.VMEM((2, CHUNK, HALF), jnp.bfloat16),
            pltpu.VMEM((2, CHUNK, HALF), jnp.bfloat16),
            pltpu.VMEM((2, CHUNK, HALF), jnp.bfloat16),
            pltpu.SemaphoreType.DMA((2,)),
            pltpu.SemaphoreType.DMA((2,)),
            pltpu.SemaphoreType.DMA((2,)),
            pltpu.SemaphoreType.DMA((2,)),
            pltpu.SemaphoreType.REGULAR,
            pltpu.SemaphoreType.REGULAR,
        ],
        compiler_params=pltpu.CompilerParams(collective_id=0),
    )(x, w_mat)


# device time: 100506 ns/iter; 3.1057x vs baseline; 1.1105x over previous
import jax
import jax.numpy as jnp
from jax import lax
from jax.experimental import pallas as pl
from jax.experimental.pallas import tpu as pltpu

N_DEV = 4
M = 2048
N = 2048
CHUNK = M // N_DEV
HALF = N // 2
SUB = 4
SROWS = CHUNK // SUB
N_HOPS = 2 * (N_DEV - 1)


def kernel(x, w_mat):
    def body(
        x_ref, w_ref, out_ref,
        comm_r, comm_l, sb_r, sb_l,
        ssem_r, rsem_r, ssem_l, rsem_l,
        credit_r, credit_l,
    ):
        my = lax.axis_index("i")
        left = lax.rem(my + N_DEV - 1, N_DEV)
        right = lax.rem(my + 1, N_DEV)

        barrier_sem = pltpu.get_barrier_semaphore()
        for nbr in (left, right):
            pl.semaphore_signal(
                barrier_sem, inc=1,
                device_id=(nbr,), device_id_type=pl.DeviceIdType.MESH,
            )
        pl.semaphore_wait(barrier_sem, 2)

        out_ref[:, :] = jnp.dot(
            x_ref[:, :], w_ref[:, :], preferred_element_type=jnp.float32
        )

        def mod4(v):
            return lax.rem(v + 2 * N_DEV, N_DEV)

        rings = [
            ("r", comm_r, sb_r, ssem_r, rsem_r, credit_r, right, left, 0),
            ("l", comm_l, sb_l, ssem_l, rsem_l, credit_l, left, right, HALF),
        ]

        def chunk_ids(col0, h):
            if col0 == 0:
                if h < N_DEV - 1:
                    return mod4(my - h), mod4(my - 1 - h)
                a = h - (N_DEV - 1)
                return mod4(my + 1 - a), mod4(my - a)
            else:
                if h < N_DEV - 1:
                    return mod4(my + h), mod4(my + 1 + h)
                a = h - (N_DEV - 1)
                return mod4(my - 1 + a), mod4(my + a)

        def rows(chunk_idx, s):
            return pl.ds(chunk_idx * CHUNK + s * SROWS, SROWS)

        def stage(ring, h, s):
            _, _, sb, _, _, _, _, _, col0 = ring
            src, _ = chunk_ids(col0, h)
            sb[h % 2, s] = out_ref[
                rows(src, s), col0:col0 + HALF
            ].astype(jnp.bfloat16)

        def consume(ring, h, s):
            comm, _, _, _, _, _, _, col0 = ring[1:]
            _, dst = chunk_ids(col0, h)
            val = comm[h % 2, s].astype(jnp.float32)
            if h < N_DEV - 1:
                out_ref[rows(dst, s), col0:col0 + HALF] += val
            else:
                out_ref[rows(dst, s), col0:col0 + HALF] = val

        rdmas = {}

        def make(ring, h, s):
            name, comm, sb, ssem, rsem, _, dst_dev, _, _ = ring
            slot = h % 2
            rd = pltpu.make_async_remote_copy(
                src_ref=sb.at[slot, s],
                dst_ref=comm.at[slot, s],
                send_sem=ssem.at[slot * SUB + s],
                recv_sem=rsem.at[slot * SUB + s],
                device_id=(dst_dev,),
                device_id_type=pl.DeviceIdType.MESH,
            )
            rdmas[(name, h, s)] = rd
            return rd

        for s in range(SUB):
            for ring in rings:
                stage(ring, 0, s)
                make(ring, 0, s).start()

        for h in range(1, N_HOPS):
            for s in range(SUB):
                for ring in rings:
                    name, _, _, _, _, credit, _, credit_dev, _ = ring
                    rdmas[(name, h - 1, s)].wait_recv()
                    consume(ring, h - 1, s)
                    if h - 1 < N_HOPS - 2:
                        pl.semaphore_signal(
                            credit, inc=1,
                            device_id=(credit_dev,),
                            device_id_type=pl.DeviceIdType.MESH,
                        )
                    if h >= 2:
                        rdmas[(name, h - 2, s)].wait_send()
                    stage(ring, h, s)
                    if h >= 2:
                        pl.semaphore_wait(credit, 1)
                    make(ring, h, s).start()

        for s in range(SUB):
            for ring in rings:
                rdmas[(ring[0], N_HOPS - 1, s)].wait_recv()
                consume(ring, N_HOPS - 1, s)
        for s in range(SUB):
            for ring in rings:
                rdmas[(ring[0], N_HOPS - 2, s)].wait_send()
                rdmas[(ring[0], N_HOPS - 1, s)].wait_send()

    return pl.pallas_call(
        body,
        out_shape=jax.ShapeDtypeStruct((M, N), jnp.float32),
        in_specs=[
            pl.BlockSpec(memory_space=pltpu.VMEM),
            pl.BlockSpec(memory_space=pltpu.VMEM),
        ],
        out_specs=pl.BlockSpec(memory_space=pltpu.VMEM),
        scratch_shapes=[
            pltpu.VMEM((2, SUB, SROWS, HALF), jnp.bfloat16),
            pltpu.VMEM((2, SUB, SROWS, HALF), jnp.bfloat16),
            pltpu.VMEM((2, SUB, SROWS, HALF), jnp.bfloat16),
            pltpu.VMEM((2, SUB, SROWS, HALF), jnp.bfloat16),
            pltpu.SemaphoreType.DMA((2 * SUB,)),
            pltpu.SemaphoreType.DMA((2 * SUB,)),
            pltpu.SemaphoreType.DMA((2 * SUB,)),
            pltpu.SemaphoreType.DMA((2 * SUB,)),
            pltpu.SemaphoreType.REGULAR,
            pltpu.SemaphoreType.REGULAR,
        ],
        compiler_params=pltpu.CompilerParams(collective_id=0),
    )(x, w_mat)


# device time: 97101 ns/iter; 3.2146x vs baseline; 1.0351x over previous
import jax
import jax.numpy as jnp
from jax import lax
from jax.experimental import pallas as pl
from jax.experimental.pallas import tpu as pltpu

N_DEV = 4
M = 2048
N = 2048
CHUNK = M // N_DEV
HALF = N // 2
SUB = 4
SROWS = CHUNK // SUB
N_HOPS = 2 * (N_DEV - 1)


def kernel(x, w_mat):
    def body(
        x_ref, w_ref, out_ref,
        comm_r, comm_l, sb_r, sb_l,
        xb, wb,
        ssem_r, rsem_r, ssem_l, rsem_l,
        credit_r, credit_l,
    ):
        my = lax.axis_index("i")
        left = lax.rem(my + N_DEV - 1, N_DEV)
        right = lax.rem(my + 1, N_DEV)

        barrier_sem = pltpu.get_barrier_semaphore()
        for nbr in (left, right):
            pl.semaphore_signal(
                barrier_sem, inc=1,
                device_id=(nbr,), device_id_type=pl.DeviceIdType.MESH,
            )
        pl.semaphore_wait(barrier_sem, 2)

        xb[:, :] = x_ref[:, :].astype(jnp.bfloat16)
        wb[:, :] = w_ref[:, :].astype(jnp.bfloat16)

        def mod4(v):
            return lax.rem(v + 2 * N_DEV, N_DEV)

        def gemm_chunk(c):
            r = pl.ds(c * CHUNK, CHUNK)
            out_ref[r, :] = jnp.dot(
                xb[r, :], wb[:, :], preferred_element_type=jnp.float32
            )

        rings = [
            ("r", comm_r, sb_r, ssem_r, rsem_r, credit_r, right, left, 0),
            ("l", comm_l, sb_l, ssem_l, rsem_l, credit_l, left, right, HALF),
        ]

        def chunk_ids(col0, h):
            if col0 == 0:
                if h < N_DEV - 1:
                    return mod4(my - h), mod4(my - 1 - h)
                a = h - (N_DEV - 1)
                return mod4(my + 1 - a), mod4(my - a)
            else:
                if h < N_DEV - 1:
                    return mod4(my + h), mod4(my + 1 + h)
                a = h - (N_DEV - 1)
                return mod4(my - 1 + a), mod4(my + a)

        def rows(chunk_idx, s):
            return pl.ds(chunk_idx * CHUNK + s * SROWS, SROWS)

        def stage(ring, h, s):
            _, _, sb, _, _, _, _, _, col0 = ring
            src, _ = chunk_ids(col0, h)
            sb[h % 2, s] = out_ref[
                rows(src, s), col0:col0 + HALF
            ].astype(jnp.bfloat16)

        def consume(ring, h, s):
            comm, _, _, _, _, _, _, col0 = ring[1:]
            _, dst = chunk_ids(col0, h)
            val = comm[h % 2, s].astype(jnp.float32)
            if h < N_DEV - 1:
                out_ref[rows(dst, s), col0:col0 + HALF] += val
            else:
                out_ref[rows(dst, s), col0:col0 + HALF] = val

        rdmas = {}

        def make(ring, h, s):
            name, comm, sb, ssem, rsem, _, dst_dev, _, _ = ring
            slot = h % 2
            rd = pltpu.make_async_remote_copy(
                src_ref=sb.at[slot, s],
                dst_ref=comm.at[slot, s],
                send_sem=ssem.at[slot * SUB + s],
                recv_sem=rsem.at[slot * SUB + s],
                device_id=(dst_dev,),
                device_id_type=pl.DeviceIdType.MESH,
            )
            rdmas[(name, h, s)] = rd
            return rd

        gemm_chunk(my)
        for s in range(SUB):
            for ring in rings:
                stage(ring, 0, s)
                make(ring, 0, s).start()
        for d in range(1, N_DEV):
            gemm_chunk(mod4(my + d))

        for h in range(1, N_HOPS):
            for s in range(SUB):
                for ring in rings:
                    name, _, _, _, _, credit, _, credit_dev, _ = ring
                    rdmas[(name, h - 1, s)].wait_recv()
                    consume(ring, h - 1, s)
                    if h - 1 < N_HOPS - 2:
                        pl.semaphore_signal(
                            credit, inc=1,
                            device_id=(credit_dev,),
                            device_id_type=pl.DeviceIdType.MESH,
                        )
                    if h >= 2:
                        rdmas[(name, h - 2, s)].wait_send()
                    stage(ring, h, s)
                    if h >= 2:
                        pl.semaphore_wait(credit, 1)
                    make(ring, h, s).start()

        for s in range(SUB):
            for ring in rings:
                rdmas[(ring[0], N_HOPS - 1, s)].wait_recv()
                consume(ring, N_HOPS - 1, s)
        for s in range(SUB):
            for ring in rings:
                rdmas[(ring[0], N_HOPS - 2, s)].wait_send()
                rdmas[(ring[0], N_HOPS - 1, s)].wait_send()

    return pl.pallas_call(
        body,
        out_shape=jax.ShapeDtypeStruct((M, N), jnp.float32),
        in_specs=[
            pl.BlockSpec(memory_space=pltpu.VMEM),
            pl.BlockSpec(memory_space=pltpu.VMEM),
        ],
        out_specs=pl.BlockSpec(memory_space=pltpu.VMEM),
        scratch_shapes=[
            pltpu.VMEM((2, SUB, SROWS, HALF), jnp.bfloat16),
            pltpu.VMEM((2, SUB, SROWS, HALF), jnp.bfloat16),
            pltpu.VMEM((2, SUB, SROWS, HALF), jnp.bfloat16),
            pltpu.VMEM((2, SUB, SROWS, HALF), jnp.bfloat16),
            pltpu.VMEM((M, M // N_DEV), jnp.bfloat16),
            pltpu.VMEM((M // N_DEV, N), jnp.bfloat16),
            pltpu.SemaphoreType.DMA((2 * SUB,)),
            pltpu.SemaphoreType.DMA((2 * SUB,)),
            pltpu.SemaphoreType.DMA((2 * SUB,)),
            pltpu.SemaphoreType.DMA((2 * SUB,)),
            pltpu.SemaphoreType.REGULAR,
            pltpu.SemaphoreType.REGULAR,
        ],
        compiler_params=pltpu.CompilerParams(collective_id=0),
    )(x, w_mat)


# device time: 91316 ns/iter; 3.4183x vs baseline; 1.0634x over previous
import jax
import jax.numpy as jnp
from jax import lax
from jax.experimental import pallas as pl
from jax.experimental.pallas import tpu as pltpu

N_DEV = 4
M = 2048
N = 2048
CHUNK = M // N_DEV
HALF = N // 2
SUB = 4
SROWS = CHUNK // SUB
N_HOPS = 2 * (N_DEV - 1)
N_COPIES = 2 * N_DEV * SUB


def kernel(x, w_mat):
    def body(
        x_ref, w_ref, out_ref,
        acc, comm_r, comm_l, sb_r, sb_l, wb,
        ssem_r, rsem_r, ssem_l, rsem_l,
        credit_r, credit_l, copy_sems,
    ):
        my = lax.axis_index("i")
        left = lax.rem(my + N_DEV - 1, N_DEV)
        right = lax.rem(my + 1, N_DEV)

        barrier_sem = pltpu.get_barrier_semaphore()
        for nbr in (left, right):
            pl.semaphore_signal(
                barrier_sem, inc=1,
                device_id=(nbr,), device_id_type=pl.DeviceIdType.MESH,
            )
        pl.semaphore_wait(barrier_sem, 2)

        wb[:, :] = w_ref[:, :].astype(jnp.bfloat16)

        def mod4(v):
            return lax.rem(v + 2 * N_DEV, N_DEV)

        def rows(chunk_idx, s):
            return pl.ds(chunk_idx * CHUNK + s * SROWS, SROWS)

        def gemm_chunk(c):
            r = pl.ds(c * CHUNK, CHUNK)
            val = jnp.dot(
                x_ref[r, :].astype(jnp.bfloat16), wb[:, :],
                preferred_element_type=jnp.float32,
            )
            acc[r, :] = val
            return val

        rings = [
            ("r", comm_r, sb_r, ssem_r, rsem_r, credit_r, right, left, 0),
            ("l", comm_l, sb_l, ssem_l, rsem_l, credit_l, left, right, HALF),
        ]

        def dst_chunk(col0, h):
            if col0 == 0:
                if h < N_DEV - 1:
                    return mod4(my - 1 - h)
                return mod4(my - (h - (N_DEV - 1)))
            else:
                if h < N_DEV - 1:
                    return mod4(my + 1 + h)
                return mod4(my + (h - (N_DEV - 1)))

        rdmas = {}

        def start(ring, h, s, src_ref):
            name, comm, _, ssem, rsem, _, dst_dev, _, _ = ring
            slot = h % 2
            rd = pltpu.make_async_remote_copy(
                src_ref=src_ref,
                dst_ref=comm.at[slot, s],
                send_sem=ssem.at[slot * SUB + s],
                recv_sem=rsem.at[slot * SUB + s],
                device_id=(dst_dev,),
                device_id_type=pl.DeviceIdType.MESH,
            )
            rdmas[(name, h, s)] = rd
            rd.start()

        def signal_credit(ring):
            _, _, _, _, _, credit, _, credit_dev, _ = ring
            pl.semaphore_signal(
                credit, inc=1,
                device_id=(credit_dev,), device_id_type=pl.DeviceIdType.MESH,
            )

        out_copies = []

        def start_out_copy(rws, cols):
            cp = pltpu.make_async_copy(
                acc.at[rws, cols],
                out_ref.at[rws, cols],
                copy_sems.at[len(out_copies)],
            )
            cp.start()
            out_copies.append(cp)

        val = gemm_chunk(my)
        for s in range(SUB):
            sl = val[s * SROWS:(s + 1) * SROWS, :]
            sb_r[0, s] = sl[:, :HALF].astype(jnp.bfloat16)
            sb_l[0, s] = sl[:, HALF:].astype(jnp.bfloat16)
            start(rings[0], 0, s, sb_r.at[0, s])
            start(rings[1], 0, s, sb_l.at[0, s])
        for d in range(1, N_DEV):
            gemm_chunk(mod4(my + d))

        for h in range(1, N_HOPS):
            hc = h - 1
            for s in range(SUB):
                for ring in rings:
                    name, comm, sb, _, _, credit, _, _, col0 = ring
                    cols = slice(col0, col0 + HALF)
                    rdmas[(name, hc, s)].wait_recv()
                    rws = rows(dst_chunk(col0, hc), s)
                    got = comm[hc % 2, s]
                    if h >= 2:
                        rdmas[(name, h - 2, s)].wait_send()
                    if hc < N_DEV - 2:
                        sb[h % 2, s] = (
                            acc[rws, cols] + got.astype(jnp.float32)
                        ).astype(jnp.bfloat16)
                        src_ref = sb.at[h % 2, s]
                        signal_credit(ring)
                    elif hc == N_DEV - 2:
                        v = acc[rws, cols] + got.astype(jnp.float32)
                        acc[rws, cols] = v
                        sb[h % 2, s] = v.astype(jnp.bfloat16)
                        src_ref = sb.at[h % 2, s]
                        signal_credit(ring)
                        start_out_copy(rws, cols)
                    else:
                        acc[rws, cols] = got.astype(jnp.float32)
                        src_ref = comm.at[hc % 2, s]
                        start_out_copy(rws, cols)
                        if h == N_HOPS - 1:
                            rdmas[(name, h - 1, s)].wait_send()
                            signal_credit(ring)
                    if h >= 2:
                        pl.semaphore_wait(credit, 1)
                    start(ring, h, s, src_ref)

        for s in range(SUB):
            for ring in rings:
                name, comm, _, _, _, _, _, _, col0 = ring
                cols = slice(col0, col0 + HALF)
                rdmas[(name, N_HOPS - 1, s)].wait_recv()
                rws = rows(dst_chunk(col0, N_HOPS - 1), s)
                acc[rws, cols] = comm[(N_HOPS - 1) % 2, s].astype(jnp.float32)
                start_out_copy(rws, cols)
        for s in range(SUB):
            for ring in rings:
                rdmas[(ring[0], N_HOPS - 1, s)].wait_send()
        for cp in out_copies:
            cp.wait()

    return pl.pallas_call(
        body,
        out_shape=jax.ShapeDtypeStruct((M, N), jnp.float32),
        in_specs=[
            pl.BlockSpec(memory_space=pltpu.VMEM),
            pl.BlockSpec(memory_space=pltpu.VMEM),
        ],
        out_specs=pl.BlockSpec(memory_space=pl.ANY),
        scratch_shapes=[
            pltpu.VMEM((M, N), jnp.float32),
            pltpu.VMEM((2, SUB, SROWS, HALF), jnp.bfloat16),
            pltpu.VMEM((2, SUB, SROWS, HALF), jnp.bfloat16),
            pltpu.VMEM((2, SUB, SROWS, HALF), jnp.bfloat16),
            pltpu.VMEM((2, SUB, SROWS, HALF), jnp.bfloat16),
            pltpu.VMEM((M // N_DEV, N), jnp.bfloat16),
            pltpu.SemaphoreType.DMA((2 * SUB,)),
            pltpu.SemaphoreType.DMA((2 * SUB,)),
            pltpu.SemaphoreType.DMA((2 * SUB,)),
            pltpu.SemaphoreType.DMA((2 * SUB,)),
            pltpu.SemaphoreType.REGULAR,
            pltpu.SemaphoreType.REGULAR,
            pltpu.SemaphoreType.DMA((N_COPIES,)),
        ],
        compiler_params=pltpu.CompilerParams(collective_id=0),
    )(x, w_mat)
